# baseline (speedup 1.0000x reference)
import functools

import jax
import jax.numpy as jnp
from jax import lax
from jax.experimental import pallas as pl
from jax.experimental.pallas import tpu as pltpu
from jax.experimental.pallas import tpu_sc as plsc

_B = 16384
_D = 32
_EPS = 1e-5
_CHUNK = 128


def _sc_gather(table, idx):
    info = plsc.get_sparse_core_info()
    nc, ns = info.num_cores, info.num_subcores
    nw = nc * ns
    bpw = _B // nw
    nchunk = bpw // _CHUNK
    rpw = bpw * _D // 128
    mesh = plsc.VectorSubcoreMesh(core_axis_name="c", subcore_axis_name="s")

    @functools.partial(
        pl.kernel,
        mesh=mesh,
        out_type=jax.ShapeDtypeStruct((_B, _D), jnp.float32),
        scratch_types=[
            pltpu.VMEM((nchunk, _CHUNK), jnp.int32),
            pltpu.VMEM((bpw, _D), jnp.float32),
            pltpu.SemaphoreType.DMA,
            pltpu.SemaphoreType.DMA,
            pltpu.SemaphoreType.DMA,
        ],
        compiler_params=pltpu.CompilerParams(
            needs_layout_passes=False, use_tc_tiling_on_sc=False,
            skip_device_barrier=True),
    )
    def k(table_h, idx_h, out_h, idx_v, rows_v, isem, gsem, wsem):
        wid = lax.axis_index("s") * nc + lax.axis_index("c")
        base = wid * bpw
        icopies = [
            pltpu.async_copy(idx_h.at[pl.ds(base + j * _CHUNK, _CHUNK)],
                             idx_v.at[j], isem)
            for j in range(nchunk)
        ]
        for c in icopies:
            c.wait()
        gathers = [
            pltpu.async_copy(table_h.at[idx_v.at[j]],
                             rows_v.at[pl.ds(j * _CHUNK, _CHUNK)], gsem)
            for j in range(nchunk)
        ]
        writes = []
        for j in range(nchunk):
            gathers[j].wait()
            writes.append(
                pltpu.async_copy(
                    rows_v.at[pl.ds(j * _CHUNK, _CHUNK)],
                    out_h.at[pl.ds(base + j * _CHUNK, _CHUNK)],
                    wsem))
        for w in writes:
            w.wait()

    return k(table, idx)


def _tc_layernorm(h128, b, gamma, beta):
    nrows = _B * _D // 128
    blk = 512

    def body(h_ref, b_ref, g_ref, be_ref, o_ref):
        m = jnp.full((_D, _D), 1.0 / _D, jnp.float32)
        x = h_ref[...] + b_ref[0]
        mean = jnp.dot(x, m, preferred_element_type=jnp.float32)
        xc = x - mean
        var = jnp.dot(xc * xc, m, preferred_element_type=jnp.float32)
        r = lax.rsqrt(var + _EPS)
        o_ref[...] = xc * r * g_ref[0] + be_ref[0]

    return pl.pallas_call(
        body,
        grid=(_B // blk,),
        in_specs=[
            pl.BlockSpec((blk, _D), lambda i: (i, 0)),
            pl.BlockSpec((1, _D), lambda i: (0, 0)),
            pl.BlockSpec((1, _D), lambda i: (0, 0)),
            pl.BlockSpec((1, _D), lambda i: (0, 0)),
        ],
        out_specs=pl.BlockSpec((blk, _D), lambda i: (i, 0)),
        out_shape=jax.ShapeDtypeStruct((_B, _D), jnp.float32),
    )(h128,
      b.reshape(1, _D),
      gamma.reshape(1, _D),
      beta.reshape(1, _D))


def kernel(species_idx, W, b, gamma, beta):
    table = W.T
    idx = species_idx.astype(jnp.int32)
    h32 = _sc_gather(table, idx)
    return _tc_layernorm(h32, b, gamma, beta)


# speedup vs baseline: 1.1924x; 1.1924x over previous
import functools

import jax
import jax.numpy as jnp
from jax import lax
from jax.experimental import pallas as pl
from jax.experimental.pallas import tpu as pltpu
from jax.experimental.pallas import tpu_sc as plsc

_B = 16384
_D = 32
_EPS = 1e-5
_CHUNK = 128


def _sc_gather(table, idx):
    info = plsc.get_sparse_core_info()
    nc, ns = info.num_cores, info.num_subcores
    nw = nc * ns
    bpw = _B // nw
    nchunk = bpw // _CHUNK
    rpw = bpw * _D // 128
    mesh = plsc.VectorSubcoreMesh(core_axis_name="c", subcore_axis_name="s")

    @functools.partial(
        pl.kernel,
        mesh=mesh,
        out_type=jax.ShapeDtypeStruct((_B * _D // 128, 128), jnp.float32),
        scratch_types=[
            pltpu.VMEM((nchunk, _CHUNK), jnp.int32),
            pltpu.VMEM((bpw, _D), jnp.float32),
            pltpu.VMEM((bpw * _D // 128, 128), jnp.float32),
            pltpu.SemaphoreType.DMA,
            pltpu.SemaphoreType.DMA,
            pltpu.SemaphoreType.DMA,
        ],
        compiler_params=pltpu.CompilerParams(
            needs_layout_passes=False, use_tc_tiling_on_sc=False,
            skip_device_barrier=True),
    )
    def k(table_h, idx_h, out_h, idx_v, rows_v, rep_v, isem, gsem, wsem):
        wid = lax.axis_index("s") * nc + lax.axis_index("c")
        base = wid * bpw
        icopies = [
            pltpu.async_copy(idx_h.at[pl.ds(base + j * _CHUNK, _CHUNK)],
                             idx_v.at[j], isem)
            for j in range(nchunk)
        ]
        for c in icopies:
            c.wait()
        gathers = [
            pltpu.async_copy(table_h.at[idx_v.at[j]],
                             rows_v.at[pl.ds(j * _CHUNK, _CHUNK)], gsem)
            for j in range(nchunk)
        ]
        rows_per_chunk = _CHUNK * _D // 128

        def repack(i, carry):
            src_row = 4 * i
            for j in range(8):
                x = rows_v[src_row + j // 2, pl.ds((j % 2) * 16, 16)]
                rep_v[i, pl.ds(j * 16, 16)] = x
            return carry

        writes = []
        for j in range(nchunk):
            gathers[j].wait()
            lax.fori_loop(j * rows_per_chunk, (j + 1) * rows_per_chunk,
                          repack, 0)
            writes.append(
                pltpu.async_copy(
                    rep_v.at[pl.ds(j * rows_per_chunk, rows_per_chunk)],
                    out_h.at[pl.ds(wid * rpw + j * rows_per_chunk,
                                   rows_per_chunk)],
                    wsem))
        for w in writes:
            w.wait()

    return k(table, idx)


def _tc_layernorm(h128, b, gamma, beta):
    nrows = _B * _D // 128
    blk = 512

    def body(h_ref, b_ref, g_ref, be_ref, o_ref):
        rows = lax.broadcasted_iota(jnp.int32, (128, 128), 0)
        cols = lax.broadcasted_iota(jnp.int32, (128, 128), 1)
        m = jnp.where((rows // _D) == (cols // _D), 1.0 / _D, 0.0)
        x = h_ref[...] + b_ref[0]
        mean = jnp.dot(x, m, preferred_element_type=jnp.float32)
        xc = x - mean
        var = jnp.dot(xc * xc, m, preferred_element_type=jnp.float32)
        r = lax.rsqrt(var + _EPS)
        o_ref[...] = xc * r * g_ref[0] + be_ref[0]

    return pl.pallas_call(
        body,
        grid=(nrows // blk,),
        in_specs=[
            pl.BlockSpec((blk, 128), lambda i: (i, 0)),
            pl.BlockSpec((1, 128), lambda i: (0, 0)),
            pl.BlockSpec((1, 128), lambda i: (0, 0)),
            pl.BlockSpec((1, 128), lambda i: (0, 0)),
        ],
        out_specs=pl.BlockSpec((blk, 128), lambda i: (i, 0)),
        out_shape=jax.ShapeDtypeStruct((nrows, 128), jnp.float32),
    )(h128,
      jnp.tile(b, 4).reshape(1, 128),
      jnp.tile(gamma, 4).reshape(1, 128),
      jnp.tile(beta, 4).reshape(1, 128))


def kernel(species_idx, W, b, gamma, beta):
    table = W.T
    idx = species_idx.astype(jnp.int32)
    h128 = _sc_gather(table, idx)
    o128 = _tc_layernorm(h128, b, gamma, beta)
    return jnp.reshape(o128, (_B, _D))
